# baseline (speedup 1.0000x reference)
import functools

import jax
import jax.numpy as jnp
from jax import lax
from jax.experimental import pallas as pl
from jax.experimental.pallas import tpu as pltpu
from jax.experimental.pallas import tpu_sc as plsc

_N = 10000
_D_IN = 128
_DH = 64
_N_PAD = 10240
_STRIPE = _N_PAD // 16
_HSH = 10016
_NW = 32
_CL = 128
_CH = 80
_CHA = 72
_CHB = 88
_EP = _NW * _CH * _CL
_EA = 16 * _CHA * _CL
_BLK = 1024
_GRID = _N_PAD // _BLK
_NBUF = 4

_mesh = plsc.VectorSubcoreMesh(core_axis_name="c", subcore_axis_name="s")


@functools.partial(
    pl.kernel,
    out_type=jax.ShapeDtypeStruct((2, _N_PAD, 16), jnp.float32),
    mesh=_mesh,
    scratch_types=[
        pltpu.VMEM((_CH, _CL), jnp.int32),
        pltpu.VMEM((_CL, 16), jnp.float32),
        pltpu.VMEM_SHARED((_N_PAD, 16), jnp.float32),
    ],
    compiler_params=pltpu.CompilerParams(use_tc_tiling_on_sc=False),
)
def _sc_degree(dst3, ones_h, zeros16, out, dst_idx, ones_v, deg_sh):
    c = lax.axis_index("c")
    s = lax.axis_index("s")
    wid = c * 16 + s
    pltpu.sync_copy(dst3.at[wid], dst_idx)
    pltpu.sync_copy(ones_h, ones_v)
    r0 = s * _STRIPE
    pltpu.sync_copy(zeros16.at[pl.ds(r0, _STRIPE)], deg_sh.at[pl.ds(r0, _STRIPE)])
    plsc.subcore_barrier()

    def body(j, carry):
        pltpu.sync_copy(ones_v, deg_sh.at[dst_idx.at[j]], add=True)
        return carry

    lax.fori_loop(0, _CH, body, 0)
    plsc.subcore_barrier()
    pltpu.sync_copy(deg_sh.at[pl.ds(r0, _STRIPE)], out.at[c, pl.ds(r0, _STRIPE)])


@functools.partial(
    pl.kernel,
    out_type=jax.ShapeDtypeStruct((2, _N_PAD, _DH), jnp.float32),
    mesh=_mesh,
    scratch_types=[
        pltpu.VMEM((2 * _NBUF, _CL), jnp.int32),
        pltpu.VMEM((2 * _NBUF, _CL), jnp.int32),
        pltpu.VMEM((_NBUF, _CL, _DH), jnp.float32),
        pltpu.VMEM_SHARED((_N_PAD, _DH), jnp.float32),
        pltpu.VMEM_SHARED((_HSH, _DH), jnp.float32),
        pltpu.SemaphoreType.DMA((_NBUF,)),
        pltpu.SemaphoreType.DMA((_NBUF,)),
        pltpu.SemaphoreType.DMA((2,)),
    ],
    compiler_params=pltpu.CompilerParams(use_tc_tiling_on_sc=False),
)
def _sc_segment_sum(srcA, dstA, srcB, dstB, hp, zeros64, out, src_ib, dst_ib,
                    rows, acc_sh, hp_sh, gsem, ssem, isem):
    c = lax.axis_index("c")
    s = lax.axis_index("s")
    h0 = s * (_HSH // 16)
    pltpu.sync_copy(hp.at[pl.ds(h0, _HSH // 16)], hp_sh.at[pl.ds(h0, _HSH // 16)])
    r0 = s * _STRIPE
    pltpu.sync_copy(zeros64.at[pl.ds(r0, _STRIPE)],
                    acc_sh.at[pl.ds(r0, _STRIPE)])
    plsc.subcore_barrier()

    def _run_chunks(src3, dst3, ch):
        nb = ch // _NBUF

        def _idx_copies(i, h):
            return (
                pltpu.make_async_copy(src3.at[s, pl.ds(i * _NBUF, _NBUF)],
                                      src_ib.at[pl.ds(h * _NBUF, _NBUF)],
                                      isem.at[0]),
                pltpu.make_async_copy(dst3.at[s, pl.ds(i * _NBUF, _NBUF)],
                                      dst_ib.at[pl.ds(h * _NBUF, _NBUF)],
                                      isem.at[1]),
            )

        def _istart(i, h):
            for cp in _idx_copies(i, h):
                cp.start()

        def _iwait(i, h):
            for cp in _idx_copies(i, h):
                cp.wait()

        def _gather(b, idx_row):
            return pltpu.async_copy(hp_sh.at[src_ib.at[idx_row]], rows.at[b],
                                    gsem.at[b])

        _istart(0, 0)
        _iwait(0, 0)
        for b in range(_NBUF):
            _gather(b, b)
        _istart(1, 1)

        def body(i, carry):
            p = lax.rem(i, 2)
            q = lax.rem(i + 1, 2)
            scat = []
            for b in range(_NBUF):
                pltpu.make_async_copy(hp_sh.at[src_ib.at[p * _NBUF + b]],
                                      rows.at[b], gsem.at[b]).wait()
                scat.append(pltpu.async_copy(
                    rows.at[b], acc_sh.at[dst_ib.at[p * _NBUF + b]],
                    ssem.at[b], add=True))

            for b in range(_NBUF):
                scat[b].wait()

            @pl.when(i < nb - 1)
            def _():
                _iwait(i + 1, q)
                for b in range(_NBUF):
                    _gather(b, q * _NBUF + b)

                @pl.when(i < nb - 2)
                def _():
                    _istart(i + 2, p)

            return carry

        lax.fori_loop(0, nb, body, 0)

    @pl.when(c == 0)
    def _():
        _run_chunks(srcA, dstA, _CHA)

    @pl.when(c == 1)
    def _():
        _run_chunks(srcB, dstB, _CHB)

    plsc.subcore_barrier()
    pltpu.sync_copy(acc_sh.at[pl.ds(r0, _STRIPE)], out.at[c, pl.ds(r0, _STRIPE)])


def _dinv_of(deg_ref):
    deg = deg_ref[0, :, 0:1] + deg_ref[1, :, 0:1] + 1.0
    return lax.rsqrt(deg)


def _tc_mm_body(x_ref, w1_ref, o_ref):
    o_ref[...] = jnp.dot(x_ref[...], w1_ref[...],
                         preferred_element_type=jnp.float32)


def _tc_scale_body(xw_ref, deg_ref, o_ref):
    o_ref[...] = xw_ref[...] * _dinv_of(deg_ref)


def _tc2_body(acc_ref, hp_ref, deg_ref, b_ref, w_ref, o_ref):
    dinv = _dinv_of(deg_ref)
    ssum = acc_ref[0] + acc_ref[1] + hp_ref[...]
    h = jnp.maximum(ssum * dinv + b_ref[...], 0.0)
    o_ref[...] = jnp.dot(h, w_ref[...],
                         preferred_element_type=jnp.float32) * dinv


def _tc3_body(acc_ref, hp_ref, deg_ref, b2_ref, wc1_ref, bc1_ref, wc2_ref,
              bc2_ref, o_ref):
    dinv = _dinv_of(deg_ref)
    ssum = acc_ref[0] + acc_ref[1] + hp_ref[...]
    h2 = jnp.maximum(ssum * dinv + b2_ref[...], 0.0)
    t = jnp.maximum(jnp.dot(h2, wc1_ref[...],
                            preferred_element_type=jnp.float32) + bc1_ref[...],
                    0.0)
    o_ref[...] = jnp.sum(t * wc2_ref[...], axis=1, keepdims=True) + bc2_ref[...]


_deg_spec = pl.BlockSpec((2, _BLK, 16), lambda i: (0, i, 0))
_acc_spec = pl.BlockSpec((2, _BLK, _DH), lambda i: (0, i, 0))
_row_spec = pl.BlockSpec((_BLK, _DH), lambda i: (i, 0))

_tc_mm = pl.pallas_call(
    _tc_mm_body,
    grid=(_GRID,),
    in_specs=[
        pl.BlockSpec((_BLK, _D_IN), lambda i: (i, 0)),
        pl.BlockSpec((_D_IN, _DH), lambda i: (0, 0)),
    ],
    out_specs=_row_spec,
    out_shape=jax.ShapeDtypeStruct((_N_PAD, _DH), jnp.float32),
)

_tc_scale = pl.pallas_call(
    _tc_scale_body,
    grid=(_GRID,),
    in_specs=[
        _row_spec,
        _deg_spec,
    ],
    out_specs=_row_spec,
    out_shape=jax.ShapeDtypeStruct((_N_PAD, _DH), jnp.float32),
)

_tc2 = pl.pallas_call(
    _tc2_body,
    grid=(_GRID,),
    in_specs=[
        _acc_spec,
        _row_spec,
        _deg_spec,
        pl.BlockSpec((1, _DH), lambda i: (0, 0)),
        pl.BlockSpec((_DH, _DH), lambda i: (0, 0)),
    ],
    out_specs=_row_spec,
    out_shape=jax.ShapeDtypeStruct((_N_PAD, _DH), jnp.float32),
)

_tc3 = pl.pallas_call(
    _tc3_body,
    grid=(_GRID,),
    in_specs=[
        _acc_spec,
        _row_spec,
        _deg_spec,
        pl.BlockSpec((1, _DH), lambda i: (0, 0)),
        pl.BlockSpec((_DH, _DH // 2), lambda i: (0, 0)),
        pl.BlockSpec((1, _DH // 2), lambda i: (0, 0)),
        pl.BlockSpec((1, _DH // 2), lambda i: (0, 0)),
        pl.BlockSpec((1, 1), lambda i: (0, 0)),
    ],
    out_specs=pl.BlockSpec((_BLK, 1), lambda i: (i, 0)),
    out_shape=jax.ShapeDtypeStruct((_N_PAD, 1), jnp.float32),
)


def kernel(x, edge_index, W1, b1, W2, b2, Wc1, bc1, Wc2, bc2):
    n = x.shape[0]
    src = edge_index[0]
    dst = edge_index[1]
    pad_e = _EP - src.shape[0]
    src_p = jnp.concatenate([src, jnp.zeros((pad_e,), jnp.int32)])
    dst_p = jnp.concatenate([dst, jnp.full((pad_e,), n, jnp.int32)])
    dst3 = dst_p.reshape(_NW, _CH, _CL)
    ones16 = jnp.ones((_CL, 16), jnp.float32)
    zeros16 = jnp.zeros((_N_PAD, 16), jnp.float32)
    zeros64 = jnp.zeros((_N_PAD, _DH), jnp.float32)

    srcA = src_p[:_EA].reshape(16, _CHA, _CL)
    dstA = dst_p[:_EA].reshape(16, _CHA, _CL)
    srcB = src_p[_EA:].reshape(16, _CHB, _CL)
    dstB = dst_p[_EA:].reshape(16, _CHB, _CL)

    deg2 = _sc_degree(dst3, ones16, zeros16)
    xw1 = _tc_mm(x, W1)
    h1p = _tc_scale(xw1, deg2)
    acc1 = _sc_segment_sum(srcA, dstA, srcB, dstB, h1p, zeros64)
    h2p = _tc2(acc1, h1p, deg2, b1.reshape(1, _DH), W2)
    acc2 = _sc_segment_sum(srcA, dstA, srcB, dstB, h2p, zeros64)
    outp = _tc3(acc2, h2p, deg2, b2.reshape(1, _DH), Wc1,
                bc1.reshape(1, _DH // 2), Wc2.reshape(1, _DH // 2),
                bc2.reshape(1, 1))
    return outp[:n]


# speedup vs baseline: 1.0483x; 1.0483x over previous
import functools

import jax
import jax.numpy as jnp
from jax import lax
from jax.experimental import pallas as pl
from jax.experimental.pallas import tpu as pltpu
from jax.experimental.pallas import tpu_sc as plsc

_N = 10000
_D_IN = 128
_DH = 64
_N_PAD = 10240
_STRIPE = _N_PAD // 16
_HSH = 10016
_NW = 32
_CL = 128
_CH = 80
_CHA = 88
_CHB = 72
_EP = _NW * _CH * _CL
_EA = 16 * _CHA * _CL
_BLK = 1024
_GRID = _N_PAD // _BLK
_NBUF = 4

_mesh = plsc.VectorSubcoreMesh(core_axis_name="c", subcore_axis_name="s")


@functools.partial(
    pl.kernel,
    out_type=jax.ShapeDtypeStruct((2, _N_PAD, 16), jnp.float32),
    mesh=_mesh,
    scratch_types=[
        pltpu.VMEM((_CH, _CL), jnp.int32),
        pltpu.VMEM((_CL, 16), jnp.float32),
        pltpu.VMEM_SHARED((_N_PAD, 16), jnp.float32),
    ],
    compiler_params=pltpu.CompilerParams(use_tc_tiling_on_sc=False),
)
def _sc_degree(dst3, ones_h, zeros16, out, dst_idx, ones_v, deg_sh):
    c = lax.axis_index("c")
    s = lax.axis_index("s")
    wid = c * 16 + s
    pltpu.sync_copy(dst3.at[wid], dst_idx)
    pltpu.sync_copy(ones_h, ones_v)
    r0 = s * _STRIPE
    pltpu.sync_copy(zeros16.at[pl.ds(r0, _STRIPE)], deg_sh.at[pl.ds(r0, _STRIPE)])
    plsc.subcore_barrier()

    def body(j, carry):
        pltpu.sync_copy(ones_v, deg_sh.at[dst_idx.at[j]], add=True)
        return carry

    lax.fori_loop(0, _CH, body, 0)
    plsc.subcore_barrier()
    pltpu.sync_copy(deg_sh.at[pl.ds(r0, _STRIPE)], out.at[c, pl.ds(r0, _STRIPE)])


@functools.partial(
    pl.kernel,
    out_type=jax.ShapeDtypeStruct((2, _N_PAD, _DH), jnp.float32),
    mesh=_mesh,
    scratch_types=[
        pltpu.VMEM((2 * _NBUF, _CL), jnp.int32),
        pltpu.VMEM((2 * _NBUF, _CL), jnp.int32),
        pltpu.VMEM((_NBUF, _CL, _DH), jnp.float32),
        pltpu.VMEM_SHARED((_N_PAD, _DH), jnp.float32),
        pltpu.VMEM_SHARED((_HSH, _DH), jnp.float32),
        pltpu.SemaphoreType.DMA((_NBUF,)),
        pltpu.SemaphoreType.DMA((_NBUF,)),
        pltpu.SemaphoreType.DMA((2,)),
    ],
    compiler_params=pltpu.CompilerParams(use_tc_tiling_on_sc=False),
)
def _sc_segment_sum(srcA, dstA, srcB, dstB, hp, zeros64, out, src_ib, dst_ib,
                    rows, acc_sh, hp_sh, gsem, ssem, isem):
    c = lax.axis_index("c")
    s = lax.axis_index("s")
    h0 = s * (_HSH // 16)
    pltpu.sync_copy(hp.at[pl.ds(h0, _HSH // 16)], hp_sh.at[pl.ds(h0, _HSH // 16)])
    r0 = s * _STRIPE
    pltpu.sync_copy(zeros64.at[pl.ds(r0, _STRIPE)],
                    acc_sh.at[pl.ds(r0, _STRIPE)])
    plsc.subcore_barrier()

    def _run_chunks(src3, dst3, ch):
        nb = ch // _NBUF

        def _idx_copies(i, h):
            return (
                pltpu.make_async_copy(src3.at[s, pl.ds(i * _NBUF, _NBUF)],
                                      src_ib.at[pl.ds(h * _NBUF, _NBUF)],
                                      isem.at[0]),
                pltpu.make_async_copy(dst3.at[s, pl.ds(i * _NBUF, _NBUF)],
                                      dst_ib.at[pl.ds(h * _NBUF, _NBUF)],
                                      isem.at[1]),
            )

        def _istart(i, h):
            for cp in _idx_copies(i, h):
                cp.start()

        def _iwait(i, h):
            for cp in _idx_copies(i, h):
                cp.wait()

        def _gather(b, idx_row):
            return pltpu.async_copy(hp_sh.at[src_ib.at[idx_row]], rows.at[b],
                                    gsem.at[b])

        _istart(0, 0)
        _iwait(0, 0)
        for b in range(_NBUF):
            _gather(b, b)
        _istart(1, 1)

        def body(i, carry):
            p = lax.rem(i, 2)
            q = lax.rem(i + 1, 2)
            scat = []
            for b in range(_NBUF):
                pltpu.make_async_copy(hp_sh.at[src_ib.at[p * _NBUF + b]],
                                      rows.at[b], gsem.at[b]).wait()
                scat.append(pltpu.async_copy(
                    rows.at[b], acc_sh.at[dst_ib.at[p * _NBUF + b]],
                    ssem.at[b], add=True))

            for b in range(_NBUF):
                scat[b].wait()

            @pl.when(i < nb - 1)
            def _():
                _iwait(i + 1, q)
                for b in range(_NBUF):
                    _gather(b, q * _NBUF + b)

                @pl.when(i < nb - 2)
                def _():
                    _istart(i + 2, p)

            return carry

        lax.fori_loop(0, nb, body, 0)

    @pl.when(c == 0)
    def _():
        _run_chunks(srcA, dstA, _CHA)

    @pl.when(c == 1)
    def _():
        _run_chunks(srcB, dstB, _CHB)

    plsc.subcore_barrier()
    pltpu.sync_copy(acc_sh.at[pl.ds(r0, _STRIPE)], out.at[c, pl.ds(r0, _STRIPE)])


def _dinv_of(deg_ref):
    deg = deg_ref[0, :, 0:1] + deg_ref[1, :, 0:1] + 1.0
    return lax.rsqrt(deg)


def _tc_mm_body(x_ref, w1_ref, o_ref):
    o_ref[...] = jnp.dot(x_ref[...], w1_ref[...],
                         preferred_element_type=jnp.float32)


def _tc_scale_body(xw_ref, deg_ref, o_ref):
    o_ref[...] = xw_ref[...] * _dinv_of(deg_ref)


def _tc2_body(acc_ref, hp_ref, deg_ref, b_ref, w_ref, o_ref):
    dinv = _dinv_of(deg_ref)
    ssum = acc_ref[0] + acc_ref[1] + hp_ref[...]
    h = jnp.maximum(ssum * dinv + b_ref[...], 0.0)
    o_ref[...] = jnp.dot(h, w_ref[...],
                         preferred_element_type=jnp.float32) * dinv


def _tc3_body(acc_ref, hp_ref, deg_ref, b2_ref, wc1_ref, bc1_ref, wc2_ref,
              bc2_ref, o_ref):
    dinv = _dinv_of(deg_ref)
    ssum = acc_ref[0] + acc_ref[1] + hp_ref[...]
    h2 = jnp.maximum(ssum * dinv + b2_ref[...], 0.0)
    t = jnp.maximum(jnp.dot(h2, wc1_ref[...],
                            preferred_element_type=jnp.float32) + bc1_ref[...],
                    0.0)
    o_ref[...] = jnp.sum(t * wc2_ref[...], axis=1, keepdims=True) + bc2_ref[...]


_deg_spec = pl.BlockSpec((2, _BLK, 16), lambda i: (0, i, 0))
_acc_spec = pl.BlockSpec((2, _BLK, _DH), lambda i: (0, i, 0))
_row_spec = pl.BlockSpec((_BLK, _DH), lambda i: (i, 0))

_tc_mm = pl.pallas_call(
    _tc_mm_body,
    grid=(_GRID,),
    in_specs=[
        pl.BlockSpec((_BLK, _D_IN), lambda i: (i, 0)),
        pl.BlockSpec((_D_IN, _DH), lambda i: (0, 0)),
    ],
    out_specs=_row_spec,
    out_shape=jax.ShapeDtypeStruct((_N_PAD, _DH), jnp.float32),
)

_tc_scale = pl.pallas_call(
    _tc_scale_body,
    grid=(_GRID,),
    in_specs=[
        _row_spec,
        _deg_spec,
    ],
    out_specs=_row_spec,
    out_shape=jax.ShapeDtypeStruct((_N_PAD, _DH), jnp.float32),
)

_tc2 = pl.pallas_call(
    _tc2_body,
    grid=(_GRID,),
    in_specs=[
        _acc_spec,
        _row_spec,
        _deg_spec,
        pl.BlockSpec((1, _DH), lambda i: (0, 0)),
        pl.BlockSpec((_DH, _DH), lambda i: (0, 0)),
    ],
    out_specs=_row_spec,
    out_shape=jax.ShapeDtypeStruct((_N_PAD, _DH), jnp.float32),
)

_tc3 = pl.pallas_call(
    _tc3_body,
    grid=(_GRID,),
    in_specs=[
        _acc_spec,
        _row_spec,
        _deg_spec,
        pl.BlockSpec((1, _DH), lambda i: (0, 0)),
        pl.BlockSpec((_DH, _DH // 2), lambda i: (0, 0)),
        pl.BlockSpec((1, _DH // 2), lambda i: (0, 0)),
        pl.BlockSpec((1, _DH // 2), lambda i: (0, 0)),
        pl.BlockSpec((1, 1), lambda i: (0, 0)),
    ],
    out_specs=pl.BlockSpec((_BLK, 1), lambda i: (i, 0)),
    out_shape=jax.ShapeDtypeStruct((_N_PAD, 1), jnp.float32),
)


def kernel(x, edge_index, W1, b1, W2, b2, Wc1, bc1, Wc2, bc2):
    n = x.shape[0]
    src = edge_index[0]
    dst = edge_index[1]
    pad_e = _EP - src.shape[0]
    src_p = jnp.concatenate([src, jnp.zeros((pad_e,), jnp.int32)])
    dst_p = jnp.concatenate([dst, jnp.full((pad_e,), n, jnp.int32)])
    dst3 = dst_p.reshape(_NW, _CH, _CL)
    ones16 = jnp.ones((_CL, 16), jnp.float32)
    zeros16 = jnp.zeros((_N_PAD, 16), jnp.float32)
    zeros64 = jnp.zeros((_N_PAD, _DH), jnp.float32)

    srcA = src_p[:_EA].reshape(16, _CHA, _CL)
    dstA = dst_p[:_EA].reshape(16, _CHA, _CL)
    srcB = src_p[_EA:].reshape(16, _CHB, _CL)
    dstB = dst_p[_EA:].reshape(16, _CHB, _CL)

    deg2 = _sc_degree(dst3, ones16, zeros16)
    xw1 = _tc_mm(x, W1)
    h1p = _tc_scale(xw1, deg2)
    acc1 = _sc_segment_sum(srcA, dstA, srcB, dstB, h1p, zeros64)
    h2p = _tc2(acc1, h1p, deg2, b1.reshape(1, _DH), W2)
    acc2 = _sc_segment_sum(srcA, dstA, srcB, dstB, h2p, zeros64)
    outp = _tc3(acc2, h2p, deg2, b2.reshape(1, _DH), Wc1,
                bc1.reshape(1, _DH // 2), Wc2.reshape(1, _DH // 2),
                bc2.reshape(1, 1))
    return outp[:n]
